# Initial kernel scaffold; baseline (speedup 1.0000x reference)
import jax
import jax.numpy as jnp
from jax.experimental import pallas as pl


def kernel(inputs, offsets, W, b):
    raise NotImplementedError("write your pallas kernel here")


# speedup vs baseline: 6.0221x; 6.0221x over previous
import functools

import numpy as np
import jax
import jax.numpy as jnp
from jax import lax
from jax.experimental import pallas as pl
from jax.experimental.pallas import tpu as pltpu
from jax.experimental.pallas import tpu_sc as plsc

KH, KW = 3, 3
N_TAP = KH * KW

_NC, _NS = 2, 16
_NW = _NC * _NS


def _grid_offset_np(h, w):
    init = np.stack(np.meshgrid(np.arange(KH), np.arange(KW), indexing="ij"))
    init = init.reshape(-1, 2)[None, None, :, :]
    init = np.tile(init, (h, w, 1, 1)).astype(np.float32)
    off0 = int((KH - 1) / 2.0)
    off1 = int((KW - 1) / 2.0)
    grid = np.meshgrid(np.arange(-off0, h - off0), np.arange(-off1, w - off1),
                       indexing="ij")
    grid = np.stack(grid, axis=-1).astype(np.float32)[:, :, None, :]
    grid = np.tile(grid, (1, 1, N_TAP, 1))
    return grid + init


def _idx_weight_kernel(h, w, oy, ox, gy, gx, boff,
                       i00, i01, i10, i11, w00, w01, w10, w11):
    cy = jnp.clip(gy[...] + oy[...], 0.0, float(h - 1))
    cx = jnp.clip(gx[...] + ox[...], 0.0, float(w - 1))
    y0f = jnp.floor(cy)
    x0f = jnp.floor(cx)
    fy = cy - y0f
    fx = cx - x0f
    y0 = y0f.astype(jnp.int32)
    x0 = x0f.astype(jnp.int32)
    y1 = jnp.ceil(cy).astype(jnp.int32)
    x1 = jnp.ceil(cx).astype(jnp.int32)
    base = boff[...]
    i00[...] = base + y0 * w + x0
    i01[...] = base + y0 * w + x1
    i10[...] = base + y1 * w + x0
    i11[...] = base + y1 * w + x1
    gy1 = 1.0 - fy
    gx1 = 1.0 - fx
    w00[...] = gy1 * gx1
    w01[...] = gy1 * fx
    w10[...] = fy * gx1
    w11[...] = fy * fx


def _make_gather_blend(m9, c_in, rows_per_worker, chunk):
    nchunk = rows_per_worker // chunk
    mesh = plsc.VectorSubcoreMesh(core_axis_name="c", subcore_axis_name="s")

    @functools.partial(
        pl.kernel,
        out_type=jax.ShapeDtypeStruct((m9, c_in), jnp.float32),
        mesh=mesh,
        scratch_types=[
            pltpu.VMEM((4 * chunk,), jnp.int32),
            pltpu.VMEM((chunk, 16), jnp.float32),
            pltpu.VMEM((4 * chunk, c_in), jnp.float32),
            pltpu.VMEM((chunk, c_in), jnp.float32),
            pltpu.SemaphoreType.DMA,
        ],
    )
    def gather_blend(table_hbm, idx_hbm, wgt_hbm, out_hbm,
                     idx_v, wgt_v, buf_v, out_v, sem):
        wid = lax.axis_index("s") * _NC + lax.axis_index("c")

        def chunk_body(ch, carry):
            row0 = wid * rows_per_worker + ch * chunk
            base4 = row0 * 4
            pltpu.sync_copy(idx_hbm.at[pl.ds(base4, 4 * chunk)], idx_v)
            pltpu.sync_copy(wgt_hbm.at[pl.ds(row0, chunk)], wgt_v)
            pltpu.async_copy(table_hbm.at[idx_v], buf_v, sem).wait()

            def row_body(r, c2):
                wv = wgt_v[r]
                w0 = wv[0]
                w1 = wv[1]
                w2 = wv[2]
                w3 = wv[3]
                for k in range(c_in // 16):
                    sl = pl.ds(k * 16, 16)
                    out_v[r, sl] = (buf_v[4 * r, sl] * w0
                                    + buf_v[4 * r + 1, sl] * w1
                                    + buf_v[4 * r + 2, sl] * w2
                                    + buf_v[4 * r + 3, sl] * w3)
                return c2

            lax.fori_loop(0, chunk, row_body, 0)
            pltpu.sync_copy(out_v, out_hbm.at[pl.ds(row0, chunk)])
            return carry

        lax.fori_loop(0, nchunk, chunk_body, 0)

    return gather_blend


def _matmul_kernel(a_ref, w_ref, b_ref, o_ref):
    o_ref[...] = (
        jnp.dot(a_ref[...], w_ref[...], preferred_element_type=jnp.float32)
        + b_ref[...]
    )


def kernel(inputs, offsets, W, b):
    bsz, h, w, c_in = inputs.shape
    n_tap, _, c_out = W.shape
    m = bsz * h * w
    m9 = m * n_tap

    grid = _grid_offset_np(h, w)
    gy = np.tile(grid[None, ..., 0], (bsz, 1, 1, 1)).reshape(-1)
    gx = np.tile(grid[None, ..., 1], (bsz, 1, 1, 1)).reshape(-1)
    boff = (np.arange(m9, dtype=np.int64) // (h * w * n_tap) * (h * w)).astype(
        np.int32)

    lanes = 128
    rows128 = m9 // lanes
    gy = jnp.asarray(gy.reshape(rows128, lanes))
    gx = jnp.asarray(gx.reshape(rows128, lanes))
    boff = jnp.asarray(boff.reshape(rows128, lanes))

    off5 = offsets.reshape(bsz, h, w, n_tap, 2)
    oy = off5[..., 0].reshape(rows128, lanes)
    ox = off5[..., 1].reshape(rows128, lanes)

    shp_i = jax.ShapeDtypeStruct((rows128, lanes), jnp.int32)
    shp_f = jax.ShapeDtypeStruct((rows128, lanes), jnp.float32)
    i00, i01, i10, i11, w00, w01, w10, w11 = pl.pallas_call(
        functools.partial(_idx_weight_kernel, h, w),
        out_shape=(shp_i, shp_i, shp_i, shp_i, shp_f, shp_f, shp_f, shp_f),
    )(oy, ox, gy, gx, boff)

    idx4 = jnp.stack([i00, i01, i10, i11], axis=-1).reshape(-1)
    wgt16 = jnp.pad(
        jnp.stack([w00, w01, w10, w11], axis=-1).reshape(m9, 4),
        ((0, 0), (0, 12)))

    table = inputs.reshape(m, c_in)
    rows_per_worker = m9 // _NW
    chunk = 32
    mapped = _make_gather_blend(m9, c_in, rows_per_worker, chunk)(
        table, idx4, wgt16)

    a2 = mapped.reshape(m, n_tap * c_in)
    wf = W.reshape(n_tap * c_in, c_out)
    b2 = b.reshape(1, c_out)
    bm = 512
    out = pl.pallas_call(
        _matmul_kernel,
        grid=(m // bm,),
        in_specs=[
            pl.BlockSpec((bm, n_tap * c_in), lambda i: (i, 0)),
            pl.BlockSpec((n_tap * c_in, c_out), lambda i: (0, 0)),
            pl.BlockSpec((1, c_out), lambda i: (0, 0)),
        ],
        out_specs=pl.BlockSpec((bm, c_out), lambda i: (i, 0)),
        out_shape=jax.ShapeDtypeStruct((m, c_out), jnp.float32),
    )(a2, wf, b2)
    return out.reshape(bsz, h, w, c_out)


# speedup vs baseline: 10.7940x; 1.7924x over previous
import functools

import numpy as np
import jax
import jax.numpy as jnp
from jax import lax
from jax.experimental import pallas as pl
from jax.experimental.pallas import tpu as pltpu
from jax.experimental.pallas import tpu_sc as plsc

KH, KW = 3, 3
N_TAP = KH * KW

_NC, _NS = 2, 16
_NW = _NC * _NS


def _grid_offset_np(h, w):
    init = np.stack(np.meshgrid(np.arange(KH), np.arange(KW), indexing="ij"))
    init = init.reshape(-1, 2)[None, None, :, :]
    init = np.tile(init, (h, w, 1, 1)).astype(np.float32)
    off0 = int((KH - 1) / 2.0)
    off1 = int((KW - 1) / 2.0)
    grid = np.meshgrid(np.arange(-off0, h - off0), np.arange(-off1, w - off1),
                       indexing="ij")
    grid = np.stack(grid, axis=-1).astype(np.float32)[:, :, None, :]
    grid = np.tile(grid, (1, 1, N_TAP, 1))
    return grid + init


def _idx_weight_kernel(h, w, oy, ox, gy, gx, boff,
                       i00, i01, i10, i11, w00, w01, w10, w11):
    cy = jnp.clip(gy[...] + oy[...], 0.0, float(h - 1))
    cx = jnp.clip(gx[...] + ox[...], 0.0, float(w - 1))
    y0f = jnp.floor(cy)
    x0f = jnp.floor(cx)
    fy = cy - y0f
    fx = cx - x0f
    y0 = y0f.astype(jnp.int32)
    x0 = x0f.astype(jnp.int32)
    y1 = jnp.ceil(cy).astype(jnp.int32)
    x1 = jnp.ceil(cx).astype(jnp.int32)
    base = boff[...]
    i00[...] = base + y0 * w + x0
    i01[...] = base + y0 * w + x1
    i10[...] = base + y1 * w + x0
    i11[...] = base + y1 * w + x1
    gy1 = 1.0 - fy
    gx1 = 1.0 - fx
    w00[...] = gy1 * gx1
    w01[...] = gy1 * fx
    w10[...] = fy * gx1
    w11[...] = fy * fx


def _make_gather(m9, c_in, rows_per_worker, chunk):
    nchunk = rows_per_worker // chunk
    mesh = plsc.VectorSubcoreMesh(core_axis_name="c", subcore_axis_name="s")

    @functools.partial(
        pl.kernel,
        out_type=jax.ShapeDtypeStruct((4 * m9, c_in), jnp.float32),
        mesh=mesh,
        scratch_types=[
            pltpu.VMEM((chunk,), jnp.int32),
            pltpu.VMEM((chunk,), jnp.int32),
            pltpu.VMEM((chunk, c_in), jnp.float32),
            pltpu.VMEM((chunk, c_in), jnp.float32),
            pltpu.SemaphoreType.DMA,
            pltpu.SemaphoreType.DMA,
            pltpu.SemaphoreType.DMA,
            pltpu.SemaphoreType.DMA,
        ],
    )
    def gather(table_hbm, idx_hbm, out_hbm,
               idx_a, idx_b, buf_a, buf_b, gsem_a, gsem_b, wsem_a, wsem_b):
        wid = lax.axis_index("s") * _NC + lax.axis_index("c")
        base = wid * rows_per_worker
        idx_refs = (idx_a, idx_b)
        buf_refs = (buf_a, buf_b)
        gsems = (gsem_a, gsem_b)
        wsems = (wsem_a, wsem_b)

        for j in range(4):
            off = j * m9 + base
            gd = [None, None]
            wd = [None, None]

            def start_gather(ch, off=off, gd=gd):
                s = ch & 1
                pltpu.sync_copy(idx_hbm.at[pl.ds(off + ch * chunk, chunk)],
                                idx_refs[s])
                gd[s] = pltpu.async_copy(table_hbm.at[idx_refs[s]],
                                         buf_refs[s], gsems[s])

            start_gather(0)
            for ch in range(nchunk):
                s = ch & 1
                if ch + 1 < nchunk:
                    if wd[1 - s] is not None:
                        wd[1 - s].wait()
                        wd[1 - s] = None
                    start_gather(ch + 1)
                gd[s].wait()
                wd[s] = pltpu.async_copy(
                    buf_refs[s],
                    out_hbm.at[pl.ds(off + ch * chunk, chunk)],
                    wsems[s])
            for s in range(2):
                if wd[s] is not None:
                    wd[s].wait()

    return gather


def _blend_matmul_kernel(n_tap, cr_ref, wg_ref, w_ref, b_ref, o_ref):
    acc = None
    for n in range(n_tap):
        mapped = (cr_ref[0, n] * wg_ref[n, :, 0:1]
                  + cr_ref[1, n] * wg_ref[n, :, 1:2]
                  + cr_ref[2, n] * wg_ref[n, :, 2:3]
                  + cr_ref[3, n] * wg_ref[n, :, 3:4])
        d = jnp.dot(mapped.astype(jnp.bfloat16), w_ref[n],
                    preferred_element_type=jnp.float32)
        acc = d if acc is None else acc + d
    o_ref[...] = acc + b_ref[...]


def kernel(inputs, offsets, W, b):
    bsz, h, w, c_in = inputs.shape
    n_tap, _, c_out = W.shape
    hw = h * w
    m = bsz * hw
    m9 = m * n_tap

    grid = _grid_offset_np(h, w)
    gy = np.tile(grid[..., 0].transpose(2, 0, 1)[:, None], (1, bsz, 1, 1))
    gx = np.tile(grid[..., 1].transpose(2, 0, 1)[:, None], (1, bsz, 1, 1))
    boff = np.tile(np.repeat(np.arange(bsz, dtype=np.int32) * hw, hw), n_tap)

    lanes = 128
    rows128 = m9 // lanes
    gy = jnp.asarray(gy.reshape(rows128, lanes))
    gx = jnp.asarray(gx.reshape(rows128, lanes))
    boff = jnp.asarray(boff.reshape(rows128, lanes))

    off5 = offsets.reshape(bsz, h, w, n_tap, 2)
    oy = jnp.transpose(off5[..., 0], (3, 0, 1, 2)).reshape(rows128, lanes)
    ox = jnp.transpose(off5[..., 1], (3, 0, 1, 2)).reshape(rows128, lanes)

    shp_i = jax.ShapeDtypeStruct((rows128, lanes), jnp.int32)
    shp_f = jax.ShapeDtypeStruct((rows128, lanes), jnp.float32)
    i00, i01, i10, i11, w00, w01, w10, w11 = pl.pallas_call(
        functools.partial(_idx_weight_kernel, h, w),
        out_shape=(shp_i, shp_i, shp_i, shp_i, shp_f, shp_f, shp_f, shp_f),
    )(oy, ox, gy, gx, boff)

    idx_all = jnp.concatenate([i00.reshape(-1), i01.reshape(-1),
                               i10.reshape(-1), i11.reshape(-1)])
    wgt9 = jnp.stack([w00, w01, w10, w11], axis=-1).reshape(n_tap, m, 4)

    table = inputs.reshape(m, c_in)
    rows_per_worker = m9 // _NW
    chunk = 128
    corners = _make_gather(m9, c_in, rows_per_worker, chunk)(table, idx_all)
    corners = corners.reshape(4, n_tap, m, c_in)

    wf = W.astype(jnp.bfloat16)
    b2 = b.reshape(1, c_out)
    lb = 128
    out = pl.pallas_call(
        functools.partial(_blend_matmul_kernel, n_tap),
        grid=(m // lb,),
        in_specs=[
            pl.BlockSpec((4, n_tap, lb, c_in), lambda i: (0, 0, i, 0)),
            pl.BlockSpec((n_tap, lb, 4), lambda i: (0, i, 0)),
            pl.BlockSpec((n_tap, c_in, c_out), lambda i: (0, 0, 0)),
            pl.BlockSpec((1, c_out), lambda i: (0, 0)),
        ],
        out_specs=pl.BlockSpec((lb, c_out), lambda i: (i, 0)),
        out_shape=jax.ShapeDtypeStruct((m, c_out), jnp.float32),
    )(corners, wgt9, wf, b2)
    return out.reshape(bsz, h, w, c_out)
